# initial kernel scaffold (unmeasured)
import functools

import jax
import jax.numpy as jnp
from jax import lax
from jax.experimental import pallas as pl
from jax.experimental.pallas import tpu as pltpu

N_DEV = 4


def kernel(x, k, Wp):
    B, S, C = x.shape
    KT = k.shape[0]
    P = Wp.shape[1]
    CHUNK = S // N_DEV
    N_HOPS = 2 * (N_DEV - 1)

    def body(x_ref, k_ref, w_ref, out_ref, recv_buf, send_sems, recv_sems):
        me = lax.axis_index("i")
        left = (me - 1) % N_DEV
        right = (me + 1) % N_DEV

        barrier_sem = pltpu.get_barrier_semaphore()
        for nbr in [left, right]:
            pl.semaphore_signal(
                barrier_sem, inc=1,
                device_id=(nbr,), device_id_type=pl.DeviceIdType.MESH,
            )
        pl.semaphore_wait(barrier_sem, 2)

        w = w_ref[...]
        for j in range(N_DEV):
            lo = j * CHUNK
            acc = x_ref[:, lo:lo + CHUNK, :] * k_ref[KT - 1, :][None, None, :]
            for t in range(KT - 1):
                sh = KT - 1 - t
                if lo - sh >= 0:
                    xs = x_ref[:, lo - sh:lo + CHUNK - sh, :]
                else:
                    xs = jnp.concatenate(
                        [jnp.zeros((B, sh, C), jnp.float32),
                         x_ref[:, 0:CHUNK - sh, :]],
                        axis=1,
                    )
                acc += xs * k_ref[t, :][None, None, :]
            a = acc * (1.0 / (1.0 + jnp.exp(-acc)))
            for b in range(B):
                out_ref[b, lo:lo + CHUNK, :] = jnp.dot(
                    a[b], w, preferred_element_type=jnp.float32
                )

        for h in range(N_DEV - 1):
            src_idx = (me - h) % N_DEV
            rdma = pltpu.make_async_remote_copy(
                src_ref=out_ref.at[:, pl.ds(src_idx * CHUNK, CHUNK), :],
                dst_ref=recv_buf.at[h],
                send_sem=send_sems.at[h],
                recv_sem=recv_sems.at[h],
                device_id=(right,),
                device_id_type=pl.DeviceIdType.MESH,
            )
            rdma.start()
            rdma.wait()
            add_idx = (me - h - 1) % N_DEV
            sl = pl.ds(add_idx * CHUNK, CHUNK)
            out_ref[:, sl, :] += recv_buf[h]

        for h in range(N_DEV - 1):
            src_idx = (me + 1 - h) % N_DEV
            sem_i = (N_DEV - 1) + h
            sl = pl.ds(src_idx * CHUNK, CHUNK)
            rdma = pltpu.make_async_remote_copy(
                src_ref=out_ref.at[:, sl, :],
                dst_ref=out_ref.at[:, sl, :],
                send_sem=send_sems.at[sem_i],
                recv_sem=recv_sems.at[sem_i],
                device_id=(right,),
                device_id_type=pl.DeviceIdType.MESH,
            )
            rdma.start()
            rdma.wait()

        @functools.partial(
            pl.run_scoped, second_barrier=pltpu.SemaphoreType.REGULAR
        )
        def _(second_barrier):
            for nbr in [left, right]:
                pl.semaphore_signal(
                    second_barrier, inc=1,
                    device_id=(nbr,), device_id_type=pl.DeviceIdType.MESH,
                )
            pl.semaphore_wait(second_barrier, 2)

    return pl.pallas_call(
        body,
        out_shape=jax.ShapeDtypeStruct((B, S, P), jnp.float32),
        in_specs=[
            pl.BlockSpec(memory_space=pltpu.VMEM),
            pl.BlockSpec(memory_space=pltpu.VMEM),
            pl.BlockSpec(memory_space=pltpu.VMEM),
        ],
        out_specs=pl.BlockSpec(memory_space=pltpu.VMEM),
        scratch_shapes=[
            pltpu.VMEM((N_DEV - 1, B, CHUNK, P), jnp.float32),
            pltpu.SemaphoreType.DMA((N_HOPS,)),
            pltpu.SemaphoreType.DMA((N_HOPS,)),
        ],
        compiler_params=pltpu.CompilerParams(collective_id=0),
    )(x, k, Wp)


# baseline (device time: 634427 ns/iter reference)
import functools

import jax
import jax.numpy as jnp
from jax import lax
from jax.experimental import pallas as pl
from jax.experimental.pallas import tpu as pltpu

N_DEV = 4


def kernel(x, k, Wp):
    B, S, C = x.shape
    KT = k.shape[0]
    P = Wp.shape[1]
    CHUNK = S // N_DEV
    SUB = 512
    HALO = 8
    N_HOPS = 2 * (N_DEV - 1)

    def body(x_ref, k_ref, w_ref, out_ref,
             xw, hw, cbuf, rbuf, send_sems, recv_sems, copy_sem):
        me = lax.axis_index("i")
        left = (me - 1) % N_DEV
        right = (me + 1) % N_DEV

        barrier_sem = pltpu.get_barrier_semaphore()
        for nbr in [left, right]:
            pl.semaphore_signal(
                barrier_sem, inc=1,
                device_id=(nbr,), device_id_type=pl.DeviceIdType.MESH,
            )
        pl.semaphore_wait(barrier_sem, 2)

        def load_chunk(cj):
            lo = cj * CHUNK
            cp = pltpu.make_async_copy(
                x_ref.at[:, pl.ds(lo, CHUNK), :], xw, copy_sem)
            cp.start()
            cp.wait()

            @pl.when(cj > 0)
            def _():
                hp = pltpu.make_async_copy(
                    x_ref.at[:, pl.ds(lo - HALO, HALO), :], hw, copy_sem)
                hp.start()
                hp.wait()

            @pl.when(cj == 0)
            def _():
                hw[...] = jnp.zeros((B, HALO, C), jnp.float32)

        def compute_chunk(step):
            for b in range(B):
                for u in range(CHUNK // SUB):
                    us = u * SUB
                    acc = xw[b, us:us + SUB, :] * k_ref[KT - 1, :][None, :]
                    for t in range(KT - 1):
                        sh = KT - 1 - t
                        if us - sh >= 0:
                            xs = xw[b, us - sh:us + SUB - sh, :]
                        else:
                            xs = jnp.concatenate(
                                [hw[b, HALO - sh:HALO, :],
                                 xw[b, 0:us + SUB - sh, :]],
                                axis=0,
                            )
                        acc += xs * k_ref[t, :][None, :]
                    a = acc * (1.0 / (1.0 + jnp.exp(-acc)))
                    r = jnp.dot(a, w_ref[...],
                                preferred_element_type=jnp.float32)
                    if step == 0:
                        cbuf[b, us:us + SUB, :] = r
                    else:
                        cbuf[b, us:us + SUB, :] = (
                            r + rbuf[step - 1, b, us:us + SUB, :])

        for s in range(N_DEV):
            cj = (me - s) % N_DEV
            load_chunk(cj)
            compute_chunk(s)
            if s < N_DEV - 1:
                rdma = pltpu.make_async_remote_copy(
                    src_ref=cbuf,
                    dst_ref=rbuf.at[s],
                    send_sem=send_sems.at[s],
                    recv_sem=recv_sems.at[s],
                    device_id=(right,),
                    device_id_type=pl.DeviceIdType.MESH,
                )
                rdma.start()
                rdma.wait()

        fin = (me + 1) % N_DEV
        local = pltpu.make_async_copy(
            cbuf, out_ref.at[:, pl.ds(fin * CHUNK, CHUNK), :], copy_sem)
        local.start()
        local.wait()

        for h in range(N_DEV - 1):
            gid = (me + 1 - h) % N_DEV
            sl = pl.ds(gid * CHUNK, CHUNK)
            sem_i = (N_DEV - 1) + h
            src = cbuf if h == 0 else out_ref.at[:, sl, :]
            rdma = pltpu.make_async_remote_copy(
                src_ref=src,
                dst_ref=out_ref.at[:, sl, :],
                send_sem=send_sems.at[sem_i],
                recv_sem=recv_sems.at[sem_i],
                device_id=(right,),
                device_id_type=pl.DeviceIdType.MESH,
            )
            rdma.start()
            rdma.wait()

        @functools.partial(
            pl.run_scoped, second_barrier=pltpu.SemaphoreType.REGULAR
        )
        def _(second_barrier):
            for nbr in [left, right]:
                pl.semaphore_signal(
                    second_barrier, inc=1,
                    device_id=(nbr,), device_id_type=pl.DeviceIdType.MESH,
                )
            pl.semaphore_wait(second_barrier, 2)

    return pl.pallas_call(
        body,
        out_shape=jax.ShapeDtypeStruct((B, S, P), jnp.float32),
        in_specs=[
            pl.BlockSpec(memory_space=pl.ANY),
            pl.BlockSpec(memory_space=pltpu.VMEM),
            pl.BlockSpec(memory_space=pltpu.VMEM),
        ],
        out_specs=pl.BlockSpec(memory_space=pl.ANY),
        scratch_shapes=[
            pltpu.VMEM((B, CHUNK, C), jnp.float32),
            pltpu.VMEM((B, HALO, C), jnp.float32),
            pltpu.VMEM((B, CHUNK, P), jnp.float32),
            pltpu.VMEM((N_DEV - 1, B, CHUNK, P), jnp.float32),
            pltpu.SemaphoreType.DMA((N_HOPS,)),
            pltpu.SemaphoreType.DMA((N_HOPS,)),
            pltpu.SemaphoreType.DMA,
        ],
        compiler_params=pltpu.CompilerParams(
            collective_id=0,
            vmem_limit_bytes=60 * 1024 * 1024,
        ),
    )(x, k, Wp)


# device time: 331261 ns/iter; 1.9152x vs baseline; 1.9152x over previous
import functools

import jax
import jax.numpy as jnp
from jax import lax
from jax.experimental import pallas as pl
from jax.experimental.pallas import tpu as pltpu

N_DEV = 4


def kernel(x, k, Wp):
    B, S, C = x.shape
    KT = k.shape[0]
    P = Wp.shape[1]
    CHUNK = S // N_DEV
    SUB = 512
    HALO = 8

    def body(x_ref, k_ref, w_ref, out_ref,
             xw, hw, cb, rb, send_sems, recv_sems, copy_sems, credit_sems):
        me = lax.axis_index("i")
        left = (me - 1) % N_DEV
        right = (me + 1) % N_DEV
        downstream = [right, left]
        upstream = [left, right]

        barrier_sem = pltpu.get_barrier_semaphore()
        for nbr in [left, right]:
            pl.semaphore_signal(
                barrier_sem, inc=1,
                device_id=(nbr,), device_id_type=pl.DeviceIdType.MESH,
            )
        pl.semaphore_wait(barrier_sem, 2)

        def load_chunk(r, cj):
            lo = cj * CHUNK
            pltpu.make_async_copy(
                x_ref.at[r, pl.ds(lo, CHUNK), :], xw.at[r], copy_sems.at[r]
            ).start()

            @pl.when(cj > 0)
            def _():
                pltpu.make_async_copy(
                    x_ref.at[r, pl.ds(lo - HALO, HALO), :], hw.at[r],
                    copy_sems.at[2 + r],
                ).start()
                pltpu.make_async_copy(
                    x_ref.at[r, pl.ds(lo - HALO, HALO), :], hw.at[r],
                    copy_sems.at[2 + r],
                ).wait()

            @pl.when(cj == 0)
            def _():
                hw[r] = jnp.zeros((HALO, C), jnp.float32)

            pltpu.make_async_copy(
                x_ref.at[r, pl.ds(lo, CHUNK), :], xw.at[r], copy_sems.at[r]
            ).wait()

        def conv_silu_dot(r, u):
            us = u * SUB
            acc = xw[r, us:us + SUB, :] * k_ref[KT - 1, :][None, :]
            for t in range(KT - 1):
                sh = KT - 1 - t
                if us - sh >= 0:
                    xs = xw[r, us - sh:us + SUB - sh, :]
                else:
                    xs = jnp.concatenate(
                        [hw[r, HALO - sh:HALO, :], xw[r, 0:us + SUB - sh, :]],
                        axis=0,
                    )
                acc += xs * k_ref[t, :][None, :]
            a = acc * (1.0 / (1.0 + jnp.exp(-acc)))
            return jnp.dot(a, w_ref[...], preferred_element_type=jnp.float32)

        def rs_rdma(r, s):
            return pltpu.make_async_remote_copy(
                src_ref=cb.at[r, s % 2],
                dst_ref=rb.at[r, s % 2],
                send_sem=send_sems.at[r, s],
                recv_sem=recv_sems.at[r, s],
                device_id=(downstream[r],),
                device_id_type=pl.DeviceIdType.MESH,
            )

        for s in range(N_DEV):
            cjs = [(me - s) % N_DEV, (me + s) % N_DEV]
            for r in (0, 1):
                load_chunk(r, cjs[r])
            vals = [[conv_silu_dot(r, u) for u in range(CHUNK // SUB)]
                    for r in (0, 1)]
            for r in (0, 1):
                if s >= 2:
                    rs_rdma(r, s - 2).wait_send()
                if s >= 1:
                    rs_rdma(r, s - 1).wait_recv()
                for u in range(CHUNK // SUB):
                    us = u * SUB
                    if s == 0:
                        cb[r, 0, us:us + SUB, :] = vals[r][u]
                    else:
                        cb[r, s % 2, us:us + SUB, :] = (
                            vals[r][u] + rb[r, (s - 1) % 2, us:us + SUB, :])
                if s == 1:
                    pl.semaphore_signal(
                        credit_sems.at[r], inc=1,
                        device_id=(upstream[r],),
                        device_id_type=pl.DeviceIdType.MESH,
                    )
                if s < N_DEV - 1:
                    if s == 2:
                        pl.semaphore_wait(credit_sems.at[r], 1)
                    rs_rdma(r, s).start()

        fins = [(me + 1) % N_DEV, (me - 1) % N_DEV]
        for r in (0, 1):
            pltpu.make_async_copy(
                cb.at[r, 1],
                out_ref.at[r, pl.ds(fins[r] * CHUNK, CHUNK), :],
                copy_sems.at[r],
            ).start()
        for r in (0, 1):
            rs_rdma(r, 2).wait_send()

        def ag_rdma(r, h):
            gid = [(me + 1 - h) % N_DEV, (me - 1 + h) % N_DEV][r]
            sl = pl.ds(gid * CHUNK, CHUNK)
            src = cb.at[r, 1] if h == 0 else out_ref.at[r, sl, :]
            return pltpu.make_async_remote_copy(
                src_ref=src,
                dst_ref=out_ref.at[r, sl, :],
                send_sem=send_sems.at[r, 3 + h],
                recv_sem=recv_sems.at[r, 3 + h],
                device_id=(downstream[r],),
                device_id_type=pl.DeviceIdType.MESH,
            )

        for h in range(N_DEV - 1):
            for r in (0, 1):
                ag_rdma(r, h).start()
            if h < N_DEV - 2:
                for r in (0, 1):
                    ag_rdma(r, h).wait_recv()

        for r in (0, 1):
            ag_rdma(r, N_DEV - 2).wait_recv()
            for h in range(N_DEV - 1):
                ag_rdma(r, h).wait_send()
            pltpu.make_async_copy(
                cb.at[r, 1],
                out_ref.at[r, pl.ds(fins[r] * CHUNK, CHUNK), :],
                copy_sems.at[r],
            ).wait()

        @functools.partial(
            pl.run_scoped, second_barrier=pltpu.SemaphoreType.REGULAR
        )
        def _(second_barrier):
            for nbr in [left, right]:
                pl.semaphore_signal(
                    second_barrier, inc=1,
                    device_id=(nbr,), device_id_type=pl.DeviceIdType.MESH,
                )
            pl.semaphore_wait(second_barrier, 2)

    return pl.pallas_call(
        body,
        out_shape=jax.ShapeDtypeStruct((B, S, P), jnp.float32),
        in_specs=[
            pl.BlockSpec(memory_space=pl.ANY),
            pl.BlockSpec(memory_space=pltpu.VMEM),
            pl.BlockSpec(memory_space=pltpu.VMEM),
        ],
        out_specs=pl.BlockSpec(memory_space=pl.ANY),
        scratch_shapes=[
            pltpu.VMEM((2, CHUNK, C), jnp.float32),
            pltpu.VMEM((2, HALO, C), jnp.float32),
            pltpu.VMEM((2, 2, CHUNK, P), jnp.float32),
            pltpu.VMEM((2, 2, CHUNK, P), jnp.float32),
            pltpu.SemaphoreType.DMA((2, 6)),
            pltpu.SemaphoreType.DMA((2, 6)),
            pltpu.SemaphoreType.DMA((4,)),
            pltpu.SemaphoreType.REGULAR((2,)),
        ],
        compiler_params=pltpu.CompilerParams(
            collective_id=0,
            vmem_limit_bytes=60 * 1024 * 1024,
        ),
    )(x, k, Wp)


# device time: 199665 ns/iter; 3.1775x vs baseline; 1.6591x over previous
import functools

import jax
import jax.numpy as jnp
from jax import lax
from jax.experimental import pallas as pl
from jax.experimental.pallas import tpu as pltpu

N_DEV = 4


def kernel(x, k, Wp):
    B, S, C = x.shape
    KT = k.shape[0]
    P = Wp.shape[1]
    CHUNK = S // N_DEV
    SUB = 512
    HALO = 8

    def body(x_ref, k_ref, w_ref, out_ref,
             xw, hw, cb, rb, ab, stage,
             send_sems, recv_sems, copy_sems, stage_sems, credit_sems):
        me = lax.axis_index("i")
        left = (me - 1) % N_DEV
        right = (me + 1) % N_DEV
        downstream = [right, left]
        upstream = [left, right]

        barrier_sem = pltpu.get_barrier_semaphore()
        for nbr in [left, right]:
            pl.semaphore_signal(
                barrier_sem, inc=1,
                device_id=(nbr,), device_id_type=pl.DeviceIdType.MESH,
            )
        pl.semaphore_wait(barrier_sem, 2)

        def load_chunk(r, cj):
            lo = cj * CHUNK
            pltpu.make_async_copy(
                x_ref.at[r, pl.ds(lo, CHUNK), :], xw.at[r], copy_sems.at[r]
            ).start()

            @pl.when(cj > 0)
            def _():
                pltpu.make_async_copy(
                    x_ref.at[r, pl.ds(lo - HALO, HALO), :], hw.at[r],
                    copy_sems.at[2 + r],
                ).start()
                pltpu.make_async_copy(
                    x_ref.at[r, pl.ds(lo - HALO, HALO), :], hw.at[r],
                    copy_sems.at[2 + r],
                ).wait()

            @pl.when(cj == 0)
            def _():
                hw[r] = jnp.zeros((HALO, C), jnp.float32)

            pltpu.make_async_copy(
                x_ref.at[r, pl.ds(lo, CHUNK), :], xw.at[r], copy_sems.at[r]
            ).wait()

        def conv_silu_dot(r, u):
            us = u * SUB
            acc = xw[r, us:us + SUB, :] * k_ref[KT - 1, :][None, :]
            for t in range(KT - 1):
                sh = KT - 1 - t
                if us - sh >= 0:
                    xs = xw[r, us - sh:us + SUB - sh, :]
                else:
                    xs = jnp.concatenate(
                        [hw[r, HALO - sh:HALO, :], xw[r, 0:us + SUB - sh, :]],
                        axis=0,
                    )
                acc += xs * k_ref[t, :][None, :]
            a = acc * (1.0 / (1.0 + jnp.exp(-acc)))
            return jnp.dot(a, w_ref[...], preferred_element_type=jnp.float32)

        def rs_rdma(r, s):
            return pltpu.make_async_remote_copy(
                src_ref=cb.at[r, s % 2],
                dst_ref=rb.at[r, s % 2],
                send_sem=send_sems.at[r, s],
                recv_sem=recv_sems.at[r, s],
                device_id=(downstream[r],),
                device_id_type=pl.DeviceIdType.MESH,
            )

        for s in range(N_DEV):
            cjs = [(me - s) % N_DEV, (me + s) % N_DEV]
            for r in (0, 1):
                load_chunk(r, cjs[r])
            vals = [[conv_silu_dot(r, u) for u in range(CHUNK // SUB)]
                    for r in (0, 1)]
            for r in (0, 1):
                if s >= 2:
                    rs_rdma(r, s - 2).wait_send()
                if s >= 1:
                    rs_rdma(r, s - 1).wait_recv()
                for u in range(CHUNK // SUB):
                    us = u * SUB
                    if s == 0:
                        cb[r, 0, us:us + SUB, :] = vals[r][u].astype(
                            jnp.bfloat16)
                    else:
                        cb[r, s % 2, us:us + SUB, :] = (
                            vals[r][u]
                            + rb[r, (s - 1) % 2, us:us + SUB, :].astype(
                                jnp.float32)
                        ).astype(jnp.bfloat16)
                if s == 1:
                    pl.semaphore_signal(
                        credit_sems.at[r], inc=1,
                        device_id=(upstream[r],),
                        device_id_type=pl.DeviceIdType.MESH,
                    )
                if s < N_DEV - 1:
                    if s == 2:
                        pl.semaphore_wait(credit_sems.at[r], 1)
                    rs_rdma(r, s).start()
        for r in (0, 1):
            rs_rdma(r, 2).wait_send()

        def ag_rdma(r, h):
            return pltpu.make_async_remote_copy(
                src_ref=cb.at[r, 1] if h == 0 else ab.at[r, h - 1],
                dst_ref=ab.at[r, h],
                send_sem=send_sems.at[r, 3 + h],
                recv_sem=recv_sems.at[r, 3 + h],
                device_id=(downstream[r],),
                device_id_type=pl.DeviceIdType.MESH,
            )

        stage_busy = [False, False]

        def stage_out(r, src_bf16, gid):
            if stage_busy[r]:
                pltpu.make_async_copy(
                    stage.at[r], out_ref.at[r, pl.ds(0, CHUNK), :],
                    stage_sems.at[r],
                ).wait()
            stage[r] = src_bf16[...].astype(jnp.float32)
            pltpu.make_async_copy(
                stage.at[r], out_ref.at[r, pl.ds(gid * CHUNK, CHUNK), :],
                stage_sems.at[r],
            ).start()
            stage_busy[r] = True

        fins = [(me + 1) % N_DEV, (me - 1) % N_DEV]
        for r in (0, 1):
            ag_rdma(r, 0).start()
        for r in (0, 1):
            stage_out(r, cb.at[r, 1], fins[r])
        for h in range(N_DEV - 1):
            gids = [(me - h) % N_DEV, (me + h) % N_DEV]
            for r in (0, 1):
                ag_rdma(r, h).wait_recv()
            if h < N_DEV - 2:
                for r in (0, 1):
                    ag_rdma(r, h + 1).start()
            for r in (0, 1):
                stage_out(r, ab.at[r, h], gids[r])

        for r in (0, 1):
            for h in range(N_DEV - 1):
                ag_rdma(r, h).wait_send()
            pltpu.make_async_copy(
                stage.at[r], out_ref.at[r, pl.ds(0, CHUNK), :],
                stage_sems.at[r],
            ).wait()

        @functools.partial(
            pl.run_scoped, second_barrier=pltpu.SemaphoreType.REGULAR
        )
        def _(second_barrier):
            for nbr in [left, right]:
                pl.semaphore_signal(
                    second_barrier, inc=1,
                    device_id=(nbr,), device_id_type=pl.DeviceIdType.MESH,
                )
            pl.semaphore_wait(second_barrier, 2)

    return pl.pallas_call(
        body,
        out_shape=jax.ShapeDtypeStruct((B, S, P), jnp.float32),
        in_specs=[
            pl.BlockSpec(memory_space=pl.ANY),
            pl.BlockSpec(memory_space=pltpu.VMEM),
            pl.BlockSpec(memory_space=pltpu.VMEM),
        ],
        out_specs=pl.BlockSpec(memory_space=pl.ANY),
        scratch_shapes=[
            pltpu.VMEM((2, CHUNK, C), jnp.float32),
            pltpu.VMEM((2, HALO, C), jnp.float32),
            pltpu.VMEM((2, 2, CHUNK, P), jnp.bfloat16),
            pltpu.VMEM((2, 2, CHUNK, P), jnp.bfloat16),
            pltpu.VMEM((2, N_DEV - 1, CHUNK, P), jnp.bfloat16),
            pltpu.VMEM((2, CHUNK, P), jnp.float32),
            pltpu.SemaphoreType.DMA((2, 6)),
            pltpu.SemaphoreType.DMA((2, 6)),
            pltpu.SemaphoreType.DMA((4,)),
            pltpu.SemaphoreType.DMA((2,)),
            pltpu.SemaphoreType.REGULAR((2,)),
        ],
        compiler_params=pltpu.CompilerParams(
            collective_id=0,
            vmem_limit_bytes=60 * 1024 * 1024,
        ),
    )(x, k, Wp)


# device time: 199253 ns/iter; 3.1840x vs baseline; 1.0021x over previous
import functools

import jax
import jax.numpy as jnp
from jax import lax
from jax.experimental import pallas as pl
from jax.experimental.pallas import tpu as pltpu

N_DEV = 4


def kernel(x, k, Wp):
    B, S, C = x.shape
    KT = k.shape[0]
    P = Wp.shape[1]
    CHUNK = S // N_DEV
    SUB = 512
    HALO = 8

    def body(x_ref, k_ref, w_ref, out_ref,
             xw, cb, rb, ab, stage,
             send_sems, recv_sems, copy_sems, stage_sems, credit_sems):
        me = lax.axis_index("i")
        left = (me - 1) % N_DEV
        right = (me + 1) % N_DEV
        downstream = [right, left]
        upstream = [left, right]

        barrier_sem = pltpu.get_barrier_semaphore()
        for nbr in [left, right]:
            pl.semaphore_signal(
                barrier_sem, inc=1,
                device_id=(nbr,), device_id_type=pl.DeviceIdType.MESH,
            )
        pl.semaphore_wait(barrier_sem, 2)

        def load_chunk(r, cj):
            lo = cj * CHUNK

            @pl.when(cj > 0)
            def _():
                pltpu.make_async_copy(
                    x_ref.at[r, pl.ds(lo - HALO, CHUNK + HALO), :], xw.at[r],
                    copy_sems.at[r],
                ).start()
                pltpu.make_async_copy(
                    x_ref.at[r, pl.ds(lo - HALO, CHUNK + HALO), :], xw.at[r],
                    copy_sems.at[r],
                ).wait()

            @pl.when(cj == 0)
            def _():
                pltpu.make_async_copy(
                    x_ref.at[r, pl.ds(0, CHUNK), :],
                    xw.at[r, pl.ds(HALO, CHUNK), :],
                    copy_sems.at[r],
                ).start()
                xw[r, 0:HALO, :] = jnp.zeros((HALO, C), jnp.float32)
                pltpu.make_async_copy(
                    x_ref.at[r, pl.ds(0, CHUNK), :],
                    xw.at[r, pl.ds(HALO, CHUNK), :],
                    copy_sems.at[r],
                ).wait()

        def conv_silu_dot(r, u):
            us = u * SUB + HALO
            acc = xw[r, us:us + SUB, :] * k_ref[KT - 1, :][None, :]
            for t in range(KT - 1):
                sh = KT - 1 - t
                acc += xw[r, us - sh:us + SUB - sh, :] * k_ref[t, :][None, :]
            a = acc * (1.0 / (1.0 + jnp.exp(-acc)))
            return jnp.dot(
                a.astype(jnp.bfloat16),
                w_ref[...].astype(jnp.bfloat16),
                preferred_element_type=jnp.float32,
            )

        def rs_rdma(r, s):
            return pltpu.make_async_remote_copy(
                src_ref=cb.at[r, s % 2],
                dst_ref=rb.at[r, s % 2],
                send_sem=send_sems.at[r, s],
                recv_sem=recv_sems.at[r, s],
                device_id=(downstream[r],),
                device_id_type=pl.DeviceIdType.MESH,
            )

        for s in range(N_DEV):
            cjs = [(me - s) % N_DEV, (me + s) % N_DEV]
            for r in (0, 1):
                load_chunk(r, cjs[r])
            vals = [[conv_silu_dot(r, u) for u in range(CHUNK // SUB)]
                    for r in (0, 1)]
            for r in (0, 1):
                if s >= 2:
                    rs_rdma(r, s - 2).wait_send()
                if s >= 1:
                    rs_rdma(r, s - 1).wait_recv()
                for u in range(CHUNK // SUB):
                    us = u * SUB
                    if s == 0:
                        cb[r, 0, us:us + SUB, :] = vals[r][u].astype(
                            jnp.bfloat16)
                    else:
                        cb[r, s % 2, us:us + SUB, :] = (
                            vals[r][u]
                            + rb[r, (s - 1) % 2, us:us + SUB, :].astype(
                                jnp.float32)
                        ).astype(jnp.bfloat16)
                if s == 1:
                    pl.semaphore_signal(
                        credit_sems.at[r], inc=1,
                        device_id=(upstream[r],),
                        device_id_type=pl.DeviceIdType.MESH,
                    )
                if s < N_DEV - 1:
                    if s == 2:
                        pl.semaphore_wait(credit_sems.at[r], 1)
                    rs_rdma(r, s).start()
        for r in (0, 1):
            rs_rdma(r, 2).wait_send()

        def ag_rdma(r, h):
            return pltpu.make_async_remote_copy(
                src_ref=cb.at[r, 1] if h == 0 else ab.at[r, h - 1],
                dst_ref=ab.at[r, h],
                send_sem=send_sems.at[r, 3 + h],
                recv_sem=recv_sems.at[r, 3 + h],
                device_id=(downstream[r],),
                device_id_type=pl.DeviceIdType.MESH,
            )

        stage_busy = [False, False]

        def stage_out(r, src_bf16, gid):
            if stage_busy[r]:
                pltpu.make_async_copy(
                    stage.at[r], out_ref.at[r, pl.ds(0, CHUNK), :],
                    stage_sems.at[r],
                ).wait()
            stage[r] = src_bf16[...].astype(jnp.float32)
            pltpu.make_async_copy(
                stage.at[r], out_ref.at[r, pl.ds(gid * CHUNK, CHUNK), :],
                stage_sems.at[r],
            ).start()
            stage_busy[r] = True

        fins = [(me + 1) % N_DEV, (me - 1) % N_DEV]
        for r in (0, 1):
            ag_rdma(r, 0).start()
        for r in (0, 1):
            stage_out(r, cb.at[r, 1], fins[r])
        for h in range(N_DEV - 1):
            gids = [(me - h) % N_DEV, (me + h) % N_DEV]
            for r in (0, 1):
                ag_rdma(r, h).wait_recv()
            if h < N_DEV - 2:
                for r in (0, 1):
                    ag_rdma(r, h + 1).start()
            for r in (0, 1):
                stage_out(r, ab.at[r, h], gids[r])

        for r in (0, 1):
            for h in range(N_DEV - 1):
                ag_rdma(r, h).wait_send()
            pltpu.make_async_copy(
                stage.at[r], out_ref.at[r, pl.ds(0, CHUNK), :],
                stage_sems.at[r],
            ).wait()

        @functools.partial(
            pl.run_scoped, second_barrier=pltpu.SemaphoreType.REGULAR
        )
        def _(second_barrier):
            for nbr in [left, right]:
                pl.semaphore_signal(
                    second_barrier, inc=1,
                    device_id=(nbr,), device_id_type=pl.DeviceIdType.MESH,
                )
            pl.semaphore_wait(second_barrier, 2)

    return pl.pallas_call(
        body,
        out_shape=jax.ShapeDtypeStruct((B, S, P), jnp.float32),
        in_specs=[
            pl.BlockSpec(memory_space=pl.ANY),
            pl.BlockSpec(memory_space=pltpu.VMEM),
            pl.BlockSpec(memory_space=pltpu.VMEM),
        ],
        out_specs=pl.BlockSpec(memory_space=pl.ANY),
        scratch_shapes=[
            pltpu.VMEM((2, CHUNK + HALO, C), jnp.float32),
            pltpu.VMEM((2, 2, CHUNK, P), jnp.bfloat16),
            pltpu.VMEM((2, 2, CHUNK, P), jnp.bfloat16),
            pltpu.VMEM((2, N_DEV - 1, CHUNK, P), jnp.bfloat16),
            pltpu.VMEM((2, CHUNK, P), jnp.float32),
            pltpu.SemaphoreType.DMA((2, 6)),
            pltpu.SemaphoreType.DMA((2, 6)),
            pltpu.SemaphoreType.DMA((4,)),
            pltpu.SemaphoreType.DMA((2,)),
            pltpu.SemaphoreType.REGULAR((2,)),
        ],
        compiler_params=pltpu.CompilerParams(
            collective_id=0,
            vmem_limit_bytes=60 * 1024 * 1024,
        ),
    )(x, k, Wp)
